# Initial kernel scaffold; baseline (speedup 1.0000x reference)
import jax
import jax.numpy as jnp
from jax.experimental import pallas as pl


def kernel(x, W, expert_bias):
    raise NotImplementedError("write your pallas kernel here")


# speedup vs baseline: 1.4124x; 1.4124x over previous
import functools

import jax
import jax.numpy as jnp
from jax.experimental import pallas as pl
from jax.experimental.pallas import tpu as pltpu

_N_EXPERTS = 64
_TOP_K = 8


def _router_block(x_ref, w_ref, b_ref, prob_ref, idx_ref, z_ref):
    i = pl.program_id(0)
    _INT_MIN = jnp.int32(-2147483648)

    logits = jax.lax.dot_general(
        x_ref[...], w_ref[...],
        (((1,), (1,)), ((), ())),
        preferred_element_type=jnp.float32,
    ) + b_ref[...]

    t = logits.shape[0]

    bits = jax.lax.bitcast_convert_type(logits, jnp.int32)
    key = jnp.where(bits >= 0, bits, _INT_MIN - bits)
    e_iota = jax.lax.broadcasted_iota(jnp.int32, (t, _N_EXPERTS), 1)
    key = (key & jnp.int32(-64)) | (jnp.int32(63) - e_iota)

    maxes = []
    for _ in range(_TOP_K):
        m = jnp.max(key, axis=1, keepdims=True)
        maxes.append(m)
        key = jnp.where(key == m, _INT_MIN, key)
    kstack = jnp.concatenate(maxes, axis=1)

    idx_ref[...] = jnp.int32(63) - (kstack & jnp.int32(63))
    sbits = jnp.where(kstack >= 0, kstack, _INT_MIN - kstack)
    scores = jax.lax.bitcast_convert_type(sbits, jnp.float32)

    m0 = scores[:, 0:1]
    p = jnp.exp(scores - m0)
    prob_ref[...] = p / jnp.sum(p, axis=1, keepdims=True)

    se = jnp.sum(jnp.exp(logits - m0), axis=1, keepdims=True)
    lse = m0 + jnp.log(se)

    @pl.when(i == 0)
    def _():
        z_ref[...] = jnp.zeros((1, 1), jnp.float32)

    z_ref[...] += jnp.sum(lse * lse, axis=0, keepdims=True)


@functools.partial(jax.jit, static_argnames=())
def kernel(x, W, expert_bias):
    b, s, d = x.shape
    n_tok = b * s
    x2 = x.reshape(n_tok, d)
    block_t = 2048
    grid = (n_tok // block_t,)

    prob, idx, zsum = pl.pallas_call(
        _router_block,
        grid=grid,
        in_specs=[
            pl.BlockSpec((block_t, d), lambda i: (i, 0)),
            pl.BlockSpec((_N_EXPERTS, d), lambda i: (0, 0)),
            pl.BlockSpec((1, _N_EXPERTS), lambda i: (0, 0)),
        ],
        out_specs=[
            pl.BlockSpec((block_t, _TOP_K), lambda i: (i, 0)),
            pl.BlockSpec((block_t, _TOP_K), lambda i: (i, 0)),
            pl.BlockSpec((1, 1), lambda i: (0, 0)),
        ],
        out_shape=[
            jax.ShapeDtypeStruct((n_tok, _TOP_K), jnp.float32),
            jax.ShapeDtypeStruct((n_tok, _TOP_K), jnp.int32),
            jax.ShapeDtypeStruct((1, 1), jnp.float32),
        ],
        compiler_params=pltpu.CompilerParams(
            dimension_semantics=("arbitrary",),
        ),
    )(x2, W, expert_bias.reshape(1, _N_EXPERTS))

    return (prob.reshape(b, s, _TOP_K),
            idx.reshape(b, s, _TOP_K),
            zsum[0, 0] / jnp.float32(n_tok))


# speedup vs baseline: 1.6840x; 1.1922x over previous
import functools

import jax
import jax.numpy as jnp
from jax.experimental import pallas as pl
from jax.experimental.pallas import tpu as pltpu

_N_EXPERTS = 64
_TOP_K = 8


def _router_block(x_ref, w_ref, b_ref, prob_ref, idx_ref, z_ref):
    i = pl.program_id(0)
    _INT_MIN = jnp.int32(-2147483648)

    logits = jax.lax.dot_general(
        w_ref[...], x_ref[...],
        (((1,), (1,)), ((), ())),
        preferred_element_type=jnp.float32,
    ) + b_ref[...]

    t = logits.shape[1]

    bits = jax.lax.bitcast_convert_type(logits, jnp.int32)
    key = jnp.where(bits >= 0, bits, _INT_MIN - bits)
    e_iota = jax.lax.broadcasted_iota(jnp.int32, (_N_EXPERTS, t), 0)
    key = (key & jnp.int32(-64)) | (jnp.int32(63) - e_iota)

    maxes = []
    for _ in range(_TOP_K):
        m = jnp.max(key, axis=0, keepdims=True)
        maxes.append(m)
        key = jnp.where(key == m, _INT_MIN, key)
    kstack = jnp.concatenate(maxes, axis=0)

    idx = jnp.int32(63) - (kstack & jnp.int32(63))
    sbits = jnp.where(kstack >= 0, kstack, _INT_MIN - kstack)
    scores = jax.lax.bitcast_convert_type(sbits, jnp.float32)

    m0 = scores[0:1, :]
    p = jnp.exp(scores - m0)
    prob = p / jnp.sum(p, axis=0, keepdims=True)

    prob_ref[...] = prob.T
    idx_ref[...] = idx.T

    se = jnp.sum(jnp.exp(logits - m0), axis=0, keepdims=True)
    lse = m0 + jnp.log(se)

    @pl.when(i == 0)
    def _():
        z_ref[...] = jnp.zeros((1, 1), jnp.float32)

    z_ref[...] += jnp.sum(lse * lse, axis=1, keepdims=True)


@functools.partial(jax.jit, static_argnames=())
def kernel(x, W, expert_bias):
    b, s, d = x.shape
    n_tok = b * s
    x2 = x.reshape(n_tok, d)
    block_t = 512
    grid = (n_tok // block_t,)

    prob, idx, zsum = pl.pallas_call(
        _router_block,
        grid=grid,
        in_specs=[
            pl.BlockSpec((block_t, d), lambda i: (i, 0)),
            pl.BlockSpec((_N_EXPERTS, d), lambda i: (0, 0)),
            pl.BlockSpec((_N_EXPERTS, 1), lambda i: (0, 0)),
        ],
        out_specs=[
            pl.BlockSpec((block_t, _TOP_K), lambda i: (i, 0)),
            pl.BlockSpec((block_t, _TOP_K), lambda i: (i, 0)),
            pl.BlockSpec((1, 1), lambda i: (0, 0)),
        ],
        out_shape=[
            jax.ShapeDtypeStruct((n_tok, _TOP_K), jnp.float32),
            jax.ShapeDtypeStruct((n_tok, _TOP_K), jnp.int32),
            jax.ShapeDtypeStruct((1, 1), jnp.float32),
        ],
        compiler_params=pltpu.CompilerParams(
            dimension_semantics=("arbitrary",),
        ),
    )(x2, W, expert_bias.reshape(_N_EXPERTS, 1))

    return (prob.reshape(b, s, _TOP_K),
            idx.reshape(b, s, _TOP_K),
            zsum[0, 0] / jnp.float32(n_tok))


# speedup vs baseline: 2.5472x; 1.5126x over previous
import functools

import jax
import jax.numpy as jnp
from jax.experimental import pallas as pl
from jax.experimental.pallas import tpu as pltpu

_N_EXPERTS = 64
_TOP_K = 8


def _router_block(x_ref, w_ref, b_ref, prob_ref, idx_ref, z_ref):
    i = pl.program_id(0)
    _INT_MIN = jnp.int32(-2147483648)

    logits = jax.lax.dot_general(
        w_ref[...], x_ref[...],
        (((1,), (1,)), ((), ())),
        preferred_element_type=jnp.float32,
    ) + b_ref[...]

    t = logits.shape[1]

    bits = jax.lax.bitcast_convert_type(logits, jnp.int32)
    key = jnp.where(bits >= 0, bits, _INT_MIN - bits)
    e_iota = jax.lax.broadcasted_iota(jnp.int32, (_N_EXPERTS, t), 0)
    key = (key & jnp.int32(-64)) | (jnp.int32(63) - e_iota)

    maxes = []
    for _ in range(_TOP_K):
        m = jnp.max(key, axis=0, keepdims=True)
        maxes.append(m)
        key = jnp.where(key == m, _INT_MIN, key)
    kstack = jnp.concatenate(maxes, axis=0)

    idx = jnp.int32(63) - (kstack & jnp.int32(63))
    sbits = jnp.where(kstack >= 0, kstack, _INT_MIN - kstack)
    scores = jax.lax.bitcast_convert_type(sbits, jnp.float32)

    m0 = scores[0:1, :]
    p = jnp.exp(scores - m0)
    prob = p / jnp.sum(p, axis=0, keepdims=True)

    prob_ref[...] = prob.T
    idx_ref[...] = idx.T

    se = jnp.sum(jnp.exp(logits - m0), axis=0, keepdims=True)
    lse = m0 + jnp.log(se)

    @pl.when(i == 0)
    def _():
        z_ref[...] = jnp.zeros((1, 1), jnp.float32)

    z_ref[...] += jnp.sum(lse * lse, axis=1, keepdims=True)


@functools.partial(jax.jit, static_argnames=())
def kernel(x, W, expert_bias):
    b, s, d = x.shape
    n_tok = b * s
    x2 = x.reshape(n_tok, d)
    block_t = 4096
    grid = (n_tok // block_t,)

    prob, idx, zsum = pl.pallas_call(
        _router_block,
        grid=grid,
        in_specs=[
            pl.BlockSpec((block_t, d), lambda i: (i, 0)),
            pl.BlockSpec((_N_EXPERTS, d), lambda i: (0, 0)),
            pl.BlockSpec((_N_EXPERTS, 1), lambda i: (0, 0)),
        ],
        out_specs=[
            pl.BlockSpec((block_t, _TOP_K), lambda i: (i, 0)),
            pl.BlockSpec((block_t, _TOP_K), lambda i: (i, 0)),
            pl.BlockSpec((1, 1), lambda i: (0, 0)),
        ],
        out_shape=[
            jax.ShapeDtypeStruct((n_tok, _TOP_K), jnp.float32),
            jax.ShapeDtypeStruct((n_tok, _TOP_K), jnp.int32),
            jax.ShapeDtypeStruct((1, 1), jnp.float32),
        ],
        compiler_params=pltpu.CompilerParams(
            dimension_semantics=("arbitrary",),
        ),
    )(x2, W, expert_bias.reshape(_N_EXPERTS, 1))

    return (prob.reshape(b, s, _TOP_K),
            idx.reshape(b, s, _TOP_K),
            zsum[0, 0] / jnp.float32(n_tok))


# speedup vs baseline: 2.6238x; 1.0301x over previous
import functools

import jax
import jax.numpy as jnp
from jax.experimental import pallas as pl
from jax.experimental.pallas import tpu as pltpu

_N_EXPERTS = 64
_TOP_K = 8


def _router_block(x_ref, w_ref, b_ref, prob_ref, idx_ref, z_ref):
    _INT_MIN = jnp.int32(-2147483648)

    bias_col = jnp.transpose(b_ref[...], (1, 0))
    logits = jax.lax.dot_general(
        w_ref[...], x_ref[0],
        (((1,), (1,)), ((), ())),
        preferred_element_type=jnp.float32,
    ) + bias_col

    t = logits.shape[1]

    bits = jax.lax.bitcast_convert_type(logits, jnp.int32)
    key = jnp.where(bits >= 0, bits, _INT_MIN - bits)
    e_iota = jax.lax.broadcasted_iota(jnp.int32, (_N_EXPERTS, t), 0)
    key = (key & jnp.int32(-64)) | (jnp.int32(63) - e_iota)

    maxes = []
    for _ in range(_TOP_K):
        m = jnp.max(key, axis=0, keepdims=True)
        maxes.append(m)
        key = jnp.where(key == m, _INT_MIN, key)
    kstack = jnp.concatenate(maxes, axis=0)

    idx = jnp.int32(63) - (kstack & jnp.int32(63))
    sbits = jnp.where(kstack >= 0, kstack, _INT_MIN - kstack)
    scores = jax.lax.bitcast_convert_type(sbits, jnp.float32)

    m0 = scores[0:1, :]
    p = jnp.exp(scores - m0)
    prob = p / jnp.sum(p, axis=0, keepdims=True)

    prob_ref[0] = prob.T
    idx_ref[0] = idx.T

    se = jnp.sum(jnp.exp(logits - m0), axis=0, keepdims=True)
    lse = m0 + jnp.log(se)

    @pl.when((pl.program_id(0) == 0) & (pl.program_id(1) == 0))
    def _():
        z_ref[...] = jnp.zeros((1, 1), jnp.float32)

    z_ref[...] += jnp.sum(lse * lse, axis=1, keepdims=True)


@functools.partial(jax.jit, static_argnames=())
def kernel(x, W, expert_bias):
    b, s, d = x.shape
    block_t = 4096
    grid = (b, s // block_t)

    prob, idx, zsum = pl.pallas_call(
        _router_block,
        grid=grid,
        in_specs=[
            pl.BlockSpec((1, block_t, d), lambda i, j: (i, j, 0)),
            pl.BlockSpec((_N_EXPERTS, d), lambda i, j: (0, 0)),
            pl.BlockSpec((1, _N_EXPERTS), lambda i, j: (0, 0)),
        ],
        out_specs=[
            pl.BlockSpec((1, block_t, _TOP_K), lambda i, j: (i, j, 0)),
            pl.BlockSpec((1, block_t, _TOP_K), lambda i, j: (i, j, 0)),
            pl.BlockSpec((1, 1), lambda i, j: (0, 0)),
        ],
        out_shape=[
            jax.ShapeDtypeStruct((b, s, _TOP_K), jnp.float32),
            jax.ShapeDtypeStruct((b, s, _TOP_K), jnp.int32),
            jax.ShapeDtypeStruct((1, 1), jnp.float32),
        ],
        compiler_params=pltpu.CompilerParams(
            dimension_semantics=("arbitrary", "arbitrary"),
        ),
    )(x, W, expert_bias.reshape(1, _N_EXPERTS))

    return (prob, idx, zsum[0, 0] / jnp.float32(b * s))


# speedup vs baseline: 4.4460x; 1.6945x over previous
import functools

import jax
import jax.numpy as jnp
from jax.experimental import pallas as pl
from jax.experimental.pallas import tpu as pltpu

_N_EXPERTS = 64
_TOP_K = 8


def _router_block(x_ref, w_ref, b_ref, prob_ref, idx_ref, z_ref):
    _INT_MIN = jnp.int32(-2147483648)

    bias_col = jnp.transpose(b_ref[...], (1, 0))
    logits = jax.lax.dot_general(
        w_ref[...], x_ref[0],
        (((1,), (1,)), ((), ())),
        preferred_element_type=jnp.float32,
    ) + bias_col

    t = logits.shape[1]

    bits = jax.lax.bitcast_convert_type(logits, jnp.int32)
    key = jnp.where(bits >= 0, bits, _INT_MIN - bits)
    e_iota = jax.lax.broadcasted_iota(jnp.int32, (_N_EXPERTS, t), 0)
    key = (key & jnp.int32(-64)) | (jnp.int32(63) - e_iota)

    maxes = []
    for _ in range(_TOP_K):
        m = jnp.max(key, axis=0, keepdims=True)
        maxes.append(m)
        key = jnp.where(key == m, _INT_MIN, key)
    kstack = jnp.concatenate(maxes, axis=0)

    idx = jnp.int32(63) - (kstack & jnp.int32(63))
    sbits = jnp.where(kstack >= 0, kstack, _INT_MIN - kstack)
    scores = jax.lax.bitcast_convert_type(sbits, jnp.float32)

    m0 = scores[0:1, :]
    p = jnp.exp(scores - m0)
    prob = p / jnp.sum(p, axis=0, keepdims=True)

    prob_ref[0] = prob
    idx_ref[0] = idx

    se = jnp.sum(jnp.exp(logits - m0), axis=0, keepdims=True)
    lse = m0 + jnp.log(se)

    @pl.when((pl.program_id(0) == 0) & (pl.program_id(1) == 0))
    def _():
        z_ref[...] = jnp.zeros((1, 1), jnp.float32)

    z_ref[...] += jnp.sum(lse * lse, axis=1, keepdims=True)


@functools.partial(jax.jit, static_argnames=())
def kernel(x, W, expert_bias):
    b, s, d = x.shape
    block_t = 4096
    grid = (b, s // block_t)

    prob, idx, zsum = pl.pallas_call(
        _router_block,
        grid=grid,
        in_specs=[
            pl.BlockSpec((1, block_t, d), lambda i, j: (i, j, 0)),
            pl.BlockSpec((_N_EXPERTS, d), lambda i, j: (0, 0)),
            pl.BlockSpec((1, _N_EXPERTS), lambda i, j: (0, 0)),
        ],
        out_specs=[
            pl.BlockSpec((1, _TOP_K, block_t), lambda i, j: (i, 0, j)),
            pl.BlockSpec((1, _TOP_K, block_t), lambda i, j: (i, 0, j)),
            pl.BlockSpec((1, 1), lambda i, j: (0, 0)),
        ],
        out_shape=[
            jax.ShapeDtypeStruct((b, _TOP_K, s), jnp.float32),
            jax.ShapeDtypeStruct((b, _TOP_K, s), jnp.int32),
            jax.ShapeDtypeStruct((1, 1), jnp.float32),
        ],
        compiler_params=pltpu.CompilerParams(
            dimension_semantics=("arbitrary", "arbitrary"),
        ),
    )(x, W, expert_bias.reshape(1, _N_EXPERTS))

    return (prob.transpose(0, 2, 1),
            idx.transpose(0, 2, 1),
            zsum[0, 0] / jnp.float32(b * s))


# speedup vs baseline: 4.4514x; 1.0012x over previous
import functools

import jax
import jax.numpy as jnp
from jax.experimental import pallas as pl
from jax.experimental.pallas import tpu as pltpu

_N_EXPERTS = 64
_TOP_K = 8


def _route_half(x2d, w, bias_col):
    _INT_MIN = jnp.int32(-2147483648)

    logits = jax.lax.dot_general(
        w, x2d,
        (((1,), (1,)), ((), ())),
        preferred_element_type=jnp.float32,
    ) + bias_col

    t = logits.shape[1]

    bits = jax.lax.bitcast_convert_type(logits, jnp.int32)
    key = jnp.where(bits >= 0, bits, _INT_MIN - bits)
    e_iota = jax.lax.broadcasted_iota(jnp.int32, (_N_EXPERTS, t), 0)
    key = (key & jnp.int32(-64)) | (jnp.int32(63) - e_iota)

    maxes = []
    for _ in range(_TOP_K):
        m = jnp.max(key, axis=0, keepdims=True)
        maxes.append(m)
        key = jnp.where(key == m, _INT_MIN, key)
    kstack = jnp.concatenate(maxes, axis=0)

    idx = jnp.int32(63) - (kstack & jnp.int32(63))
    sbits = jnp.where(kstack >= 0, kstack, _INT_MIN - kstack)
    scores = jax.lax.bitcast_convert_type(sbits, jnp.float32)

    m0 = scores[0:1, :]
    p = jnp.exp(scores - m0)
    prob = p / jnp.sum(p, axis=0, keepdims=True)

    se = jnp.sum(jnp.exp(logits - m0), axis=0, keepdims=True)
    lse = m0 + jnp.log(se)
    return prob, idx, jnp.sum(lse * lse, axis=1, keepdims=True)


def _router_block(xa_ref, xb_ref, w_ref, b_ref, prob_ref, idx_ref, z_ref):
    bias_col = jnp.transpose(b_ref[...], (1, 0))
    w = w_ref[...]

    prob_a, idx_a, z_a = _route_half(xa_ref[0], w, bias_col)
    prob_b, idx_b, z_b = _route_half(xb_ref[0], w, bias_col)

    half = prob_a.shape[1]
    prob_ref[0, :, :half] = prob_a
    prob_ref[0, :, half:] = prob_b
    idx_ref[0, :, :half] = idx_a
    idx_ref[0, :, half:] = idx_b

    @pl.when((pl.program_id(0) == 0) & (pl.program_id(1) == 0))
    def _():
        z_ref[...] = jnp.zeros((1, 1), jnp.float32)

    z_ref[...] += z_a + z_b


@functools.partial(jax.jit, static_argnames=())
def kernel(x, W, expert_bias):
    b, s, d = x.shape
    block_t = 4096
    grid = (b, s // block_t)

    prob, idx, zsum = pl.pallas_call(
        _router_block,
        grid=grid,
        in_specs=[
            pl.BlockSpec((1, block_t // 2, d), lambda i, j: (i, 2 * j, 0)),
            pl.BlockSpec((1, block_t // 2, d), lambda i, j: (i, 2 * j + 1, 0)),
            pl.BlockSpec((_N_EXPERTS, d), lambda i, j: (0, 0)),
            pl.BlockSpec((1, _N_EXPERTS), lambda i, j: (0, 0)),
        ],
        out_specs=[
            pl.BlockSpec((1, _TOP_K, block_t), lambda i, j: (i, 0, j)),
            pl.BlockSpec((1, _TOP_K, block_t), lambda i, j: (i, 0, j)),
            pl.BlockSpec((1, 1), lambda i, j: (0, 0)),
        ],
        out_shape=[
            jax.ShapeDtypeStruct((b, _TOP_K, s), jnp.float32),
            jax.ShapeDtypeStruct((b, _TOP_K, s), jnp.int32),
            jax.ShapeDtypeStruct((1, 1), jnp.float32),
        ],
        compiler_params=pltpu.CompilerParams(
            dimension_semantics=("arbitrary", "arbitrary"),
        ),
    )(x, x, W, expert_bias.reshape(1, _N_EXPERTS))

    return (prob.transpose(0, 2, 1),
            idx.transpose(0, 2, 1),
            zsum[0, 0] / jnp.float32(b * s))
